# baseline (speedup 1.0000x reference)
import jax
import jax.numpy as jnp
from jax.experimental import pallas as pl
from jax.experimental.pallas import tpu as pltpu

VOCAB = 32768
HIDDEN = 2048
NUM_EXPERTS = 16
TOP_K = 2
TOKENS = 512
EXPERT_VOCAB = VOCAB // NUM_EXPERTS
NTILES = TOKENS // 128
VSPLIT = 2
EVBLK = EXPERT_VOCAB // VSPLIT


def _routing_body(x_ref, rw_ref, out_ref):
    x = x_ref[...]
    rw = rw_ref[...]
    logits = jnp.dot(x, rw.T, preferred_element_type=jnp.float32)
    m = jnp.max(logits, axis=1, keepdims=True)
    w = jnp.exp(logits - m)
    w = w / jnp.sum(w, axis=1, keepdims=True)
    a1 = jnp.argmax(w, axis=1)
    eids = jax.lax.broadcasted_iota(jnp.int32, (TOKENS, NUM_EXPERTS), 1)
    w2 = jnp.where(eids == a1[:, None], -jnp.inf, w)
    a2 = jnp.argmax(w2, axis=1)
    zero = jnp.zeros((TOKENS, 6), jnp.int32)
    out_ref[...] = jnp.concatenate(
        [a1[:, None].astype(jnp.int32), a2[:, None].astype(jnp.int32), zero],
        axis=1)


def _main_body(tok_ref, start_ref, cnt_ref, x_ref, w_ref, out_ref,
               xg_ref, y_ref):
    e = pl.program_id(0)
    v = pl.program_id(1)
    st = start_ref[e]
    cn = cnt_ref[e]
    out_ref[...] = jnp.full((TOKENS, EVBLK), -jnp.inf, jnp.float32)
    iota8 = jax.lax.broadcasted_iota(jnp.int32, (8, 1), 0)

    @pl.when(v == 0)
    def _gather():
        def g_body(d, carry):
            rows = []
            for j in range(8):
                t = tok_ref[st + d * 8 + j]
                src = x_ref[pl.ds(pl.multiple_of((t // 8) * 8, 8), 8), :]
                rows.append(jnp.sum(jnp.where(iota8 == t % 8, src, 0.0),
                                    axis=0, keepdims=True))
            xg_ref[pl.ds(pl.multiple_of(d * 8, 8), 8), :] = jnp.concatenate(
                rows, axis=0)
            return carry

        jax.lax.fori_loop(0, (cn + 7) // 8, g_body, 0)

    w2d = w_ref[0]
    for t in range(NTILES):
        @pl.when(t * 128 < cn)
        def _tile(t=t):
            y_ref[pl.ds(t * 128, 128), :] = jnp.dot(
                xg_ref[pl.ds(t * 128, 128), :], w2d.T,
                preferred_element_type=jnp.float32)

    def s_body(r, carry):
        t = tok_ref[st + r]
        yg = y_ref[pl.ds(pl.multiple_of((r // 8) * 8, 8), 8), :]
        yrow = jnp.sum(jnp.where(iota8 == r % 8, yg, 0.0),
                       axis=0, keepdims=True)
        ob = pl.multiple_of((t // 8) * 8, 8)
        og = out_ref[pl.ds(ob, 8), :]
        out_ref[pl.ds(ob, 8), :] = jnp.where(iota8 == t % 8, yrow, og)
        return carry

    jax.lax.fori_loop(0, cn, s_body, 0)


def kernel(hidden_states, expert_weight, router_weight):
    ids8 = pl.pallas_call(
        _routing_body,
        in_specs=[
            pl.BlockSpec((TOKENS, HIDDEN), lambda: (0, 0)),
            pl.BlockSpec((NUM_EXPERTS, HIDDEN), lambda: (0, 0)),
        ],
        out_specs=pl.BlockSpec((TOKENS, 8), lambda: (0, 0)),
        out_shape=jax.ShapeDtypeStruct((TOKENS, 8), jnp.int32),
    )(hidden_states, router_weight)

    eids = ids8[:, :TOP_K].reshape(-1)
    order = jnp.argsort(eids, stable=True)
    tok_sorted = (order // TOP_K).astype(jnp.int32)
    counts = jnp.sum(eids[:, None] == jnp.arange(NUM_EXPERTS)[None, :],
                     axis=0).astype(jnp.int32)
    start = (jnp.cumsum(counts) - counts).astype(jnp.int32)
    tok_pad = jnp.concatenate([tok_sorted, jnp.zeros((16,), jnp.int32)])

    grid_spec = pltpu.PrefetchScalarGridSpec(
        num_scalar_prefetch=3,
        grid=(NUM_EXPERTS, VSPLIT),
        in_specs=[
            pl.BlockSpec((TOKENS, HIDDEN), lambda e, v, tok, st, cn: (0, 0)),
            pl.BlockSpec((1, EVBLK, HIDDEN),
                         lambda e, v, tok, st, cn: (e, v, 0)),
        ],
        out_specs=pl.BlockSpec((TOKENS, EVBLK),
                               lambda e, v, tok, st, cn: (0, e * VSPLIT + v)),
        scratch_shapes=[
            pltpu.VMEM((TOKENS, HIDDEN), jnp.float32),
            pltpu.VMEM((TOKENS, EVBLK), jnp.float32),
        ],
    )
    return pl.pallas_call(
        _main_body,
        grid_spec=grid_spec,
        out_shape=jax.ShapeDtypeStruct((TOKENS, VOCAB), jnp.float32),
    )(tok_pad, start, counts, hidden_states, expert_weight)


# speedup vs baseline: 1.4806x; 1.4806x over previous
import jax
import jax.numpy as jnp
from jax.experimental import pallas as pl
from jax.experimental.pallas import tpu as pltpu

VOCAB = 32768
HIDDEN = 2048
NUM_EXPERTS = 16
TOKENS = 512
EXPERT_VOCAB = VOCAB // NUM_EXPERTS


def _moe_head_body(x_ref, w_ref, rw_ref, out_ref):
    e = pl.program_id(0)
    x = x_ref[...]
    logits = jnp.dot(x, rw_ref[...].T, preferred_element_type=jnp.float32)
    m = jnp.max(logits, axis=1, keepdims=True)
    w = jnp.exp(logits - m)
    w = w / jnp.sum(w, axis=1, keepdims=True)
    a1 = jnp.argmax(w, axis=1)
    eids = jax.lax.broadcasted_iota(jnp.int32, (TOKENS, NUM_EXPERTS), 1)
    w2 = jnp.where(eids == a1[:, None], -jnp.inf, w)
    a2 = jnp.argmax(w2, axis=1)
    selcol = ((a1 == e) | (a2 == e))[:, None]

    xb = x.astype(jnp.bfloat16)
    wb = w_ref[0].astype(jnp.bfloat16)
    acc = jnp.dot(xb, wb.T, preferred_element_type=jnp.float32)
    out_ref[...] = jnp.where(selcol, acc, -jnp.inf)


def kernel(hidden_states, expert_weight, router_weight):
    return pl.pallas_call(
        _moe_head_body,
        grid=(NUM_EXPERTS,),
        in_specs=[
            pl.BlockSpec((TOKENS, HIDDEN), lambda e: (0, 0)),
            pl.BlockSpec((1, EXPERT_VOCAB, HIDDEN), lambda e: (e, 0, 0)),
            pl.BlockSpec((NUM_EXPERTS, HIDDEN), lambda e: (0, 0)),
        ],
        out_specs=pl.BlockSpec((TOKENS, EXPERT_VOCAB), lambda e: (0, e)),
        out_shape=jax.ShapeDtypeStruct((TOKENS, VOCAB), jnp.float32),
        compiler_params=pltpu.CompilerParams(
            dimension_semantics=("parallel",)),
    )(hidden_states, expert_weight, router_weight)
